# baseline (speedup 1.0000x reference)
import jax
import jax.numpy as jnp
from jax.experimental import pallas as pl

N = 10000
E = 160000
H = 256
NR2 = 200

_SLOPE = (1.0 / 8.0 + 1.0 / 3.0) / 2.0
_EPS = 1e-12

_BR = 1000


def _l2norm_body(x_ref, o_ref):
    x = x_ref[...]
    n = jnp.sqrt(jnp.sum(x * x, axis=1, keepdims=True))
    o_ref[...] = x / jnp.maximum(n, _EPS)


def _l2norm(x):
    return pl.pallas_call(
        _l2norm_body,
        grid=(N // _BR,),
        in_specs=[pl.BlockSpec((_BR, H), lambda i: (i, 0))],
        out_specs=pl.BlockSpec((_BR, H), lambda i: (i, 0)),
        out_shape=jax.ShapeDtypeStruct((N, H), jnp.float32),
    )(x)


def _gru(x, h, W_ih, W_hh, b_ih, b_hh):
    gi = x @ W_ih.T + b_ih
    gh = h @ W_hh.T + b_hh
    r = jax.nn.sigmoid(gi[:, :H] + gh[:, :H])
    z = jax.nn.sigmoid(gi[:, H:2 * H] + gh[:, H:2 * H])
    n = jnp.tanh(gi[:, 2 * H:] + r * gh[:, 2 * H:])
    return (1.0 - z) * n + z * h


def _norm(x):
    n = jnp.sqrt(jnp.sum(x * x, axis=-1, keepdims=True))
    return x / jnp.maximum(n, _EPS)


def kernel(edge_index, etype, r_to_e, rel_seg, dynamic_emb, emb_rel,
           W_ih_r, W_hh_r, b_ih_r, b_hh_r,
           W_ih_e, W_hh_e, b_ih_e, b_hh_e,
           W_n1, W_s1, W_n2, W_s2,
           time_gate_weight, time_gate_bias):
    h = _l2norm(dynamic_emb)
    src = edge_index[0]
    dst = edge_index[1]

    xsum = jnp.zeros((NR2, H), jnp.float32).at[rel_seg].add(h[r_to_e])
    xcnt = jnp.zeros((NR2, 1), jnp.float32).at[rel_seg].add(1.0)
    x_input = xsum / jnp.maximum(xcnt, 1.0)
    x_cat = jnp.concatenate([emb_rel, x_input], axis=1)
    h0 = _norm(_gru(x_cat, emb_rel, W_ih_r, W_hh_r, b_ih_r, b_hh_r))

    c2h0 = jnp.zeros((N, H), jnp.float32).at[dst].add(h0[etype])
    deg = jnp.zeros((N, 1), jnp.float32).at[dst].add(1.0)
    inv = 1.0 / jnp.maximum(deg, 1.0)

    cur = h
    for W_n, W_s in ((W_n1, W_s1), (W_n2, W_s2)):
        S = jnp.zeros((N, H), jnp.float32).at[dst].add(cur[src])
        cur = jnp.where((y := ((S + c2h0) * inv) @ W_n + cur @ W_s) >= 0,
                        y, _SLOPE * y)
    cur = _norm(cur)
    hn = _norm(_gru(cur, h, W_ih_e, W_hh_e, b_ih_e, b_hh_e))
    gate = jax.nn.sigmoid(hn @ time_gate_weight + time_gate_bias)
    return gate * hn + (1.0 - gate) * h


# speedup vs baseline: 2.0963x; 2.0963x over previous
import functools

import jax
import jax.numpy as jnp
from jax import lax
from jax.experimental import pallas as pl
from jax.experimental.pallas import tpu as pltpu
from jax.experimental.pallas import tpu_sc as plsc

N = 10000
E = 160000
H = 256
NR2 = 200

HW = 128
NC = 2
NS = 16
CH = 128
EP = NC * NS * CH * 40
RD = 10112
RR = 256

_SLOPE = (1.0 / 8.0 + 1.0 / 3.0) / 2.0
_EPS = 1e-12

_mesh = plsc.VectorSubcoreMesh(core_axis_name="c", subcore_axis_name="s")


def _fill(buf, nrows, val):
    v = jnp.full((16,), val, jnp.float32)

    def fr(j, _):
        def fc(t, _):
            buf[j, pl.ds(t * 16, 16)] = v
            return 0
        lax.fori_loop(0, HW // 16, fc, 0)
        return 0
    lax.fori_loop(0, nrows, fr, 0)


def _zero_slab(slab, zbuf, s, rpt):
    full, rem = rpt // CH, rpt % CH

    def za(j, _):
        pltpu.sync_copy(zbuf, slab.at[pl.ds(s * rpt + j * CH, CH)])
        return 0
    if full:
        lax.fori_loop(0, full, za, 0)
    if rem:
        pltpu.sync_copy(zbuf.at[pl.ds(0, rem)],
                        slab.at[pl.ds(s * rpt + full * CH, rem)])


def _copy_out(slab, out, c, s, rpt):
    full, rem = rpt // CH, rpt % CH

    def ca(j, _):
        r = s * rpt + j * CH
        pltpu.sync_copy(slab.at[pl.ds(r, CH)], out.at[c, pl.ds(r, CH)])
        return 0
    if full:
        lax.fori_loop(0, full, ca, 0)
    if rem:
        r = s * rpt + full * CH
        pltpu.sync_copy(slab.at[pl.ds(r, rem)], out.at[c, pl.ds(r, rem)])


def _rowscat_body(nrows, cnt, *refs):
    if cnt:
        (tbl, gidx, sidx, osum, ocnt,
         gi, si, rows, obuf, acc, cacc, sem) = refs
    else:
        (tbl, gidx, sidx, osum, gi, si, rows, acc, sem) = refs

    c = lax.axis_index("c")
    s = lax.axis_index("s")
    rpt = nrows // NS

    _fill(rows, CH, 0.0)
    _zero_slab(acc, rows, s, rpt)
    if cnt:
        _zero_slab(cacc, rows, s, rpt)
        _fill(obuf, CH, 1.0)

    plsc.subcore_barrier()

    npt = EP // NS
    nch = npt // CH

    def chunk(i, _):
        off = s * npt + i * CH
        pltpu.sync_copy(gidx.at[c, pl.ds(off, CH)], gi)
        pltpu.sync_copy(sidx.at[pl.ds(off, CH)], si)
        pltpu.async_copy(tbl.at[gi], rows, sem).wait()
        pltpu.sync_copy(rows, acc.at[si], add=True)
        if cnt:
            @pl.when(c == 0)
            def _():
                pltpu.sync_copy(obuf, cacc.at[si], add=True)
        return 0
    lax.fori_loop(0, nch, chunk, 0)

    plsc.subcore_barrier()

    _copy_out(acc, osum, c, s, rpt)
    if cnt:
        @pl.when(c == 0)
        def _():
            _copy_out(cacc, ocnt, 0, s, rpt)


def _make_rowscat(nrows, cnt):
    scratch = [
        pltpu.VMEM((CH,), jnp.int32),
        pltpu.VMEM((CH,), jnp.int32),
        pltpu.VMEM((CH, HW), jnp.float32),
    ]
    if cnt:
        scratch.append(pltpu.VMEM((CH, HW), jnp.float32))
    scratch.append(pltpu.VMEM_SHARED((nrows, HW), jnp.float32))
    if cnt:
        scratch.append(pltpu.VMEM_SHARED((nrows, HW), jnp.float32))
    scratch.append(pltpu.SemaphoreType.DMA)

    out_type = [jax.ShapeDtypeStruct((NC, nrows, HW), jnp.float32)]
    if cnt:
        out_type.append(jax.ShapeDtypeStruct((1, nrows, HW), jnp.float32))

    @jax.jit
    def run(tbl, gidx, sidx):
        k = pl.kernel(
            functools.partial(_rowscat_body, nrows, cnt),
            out_type=tuple(out_type) if cnt else out_type[0],
            mesh=_mesh,
            scratch_types=scratch,
        )
        return k(tbl, gidx, sidx)
    return run


_scat_rel = _make_rowscat(RR, True)
_scat_dst = _make_rowscat(RD, False)


def _deg_body(sidx, ocnt, si, obuf, acc):
    c = lax.axis_index("c")
    s = lax.axis_index("s")
    rpt = RD // NS

    _fill(obuf, CH, 0.0)
    _zero_slab(acc, obuf, s, rpt)
    _fill(obuf, CH, 1.0)

    plsc.subcore_barrier()

    ep2 = EP // NC
    npt = ep2 // NS
    nch = npt // CH

    def chunk(i, _):
        off = c * ep2 + s * npt + i * CH
        pltpu.sync_copy(sidx.at[pl.ds(off, CH)], si)
        pltpu.sync_copy(obuf, acc.at[si], add=True)
        return 0
    lax.fori_loop(0, nch, chunk, 0)

    plsc.subcore_barrier()
    _copy_out(acc, ocnt, c, s, rpt)


@jax.jit
def _scat_deg(sidx):
    k = pl.kernel(
        _deg_body,
        out_type=jax.ShapeDtypeStruct((NC, RD, HW), jnp.float32),
        mesh=_mesh,
        scratch_types=[
            pltpu.VMEM((CH,), jnp.int32),
            pltpu.VMEM((CH, HW), jnp.float32),
            pltpu.VMEM_SHARED((RD, HW), jnp.float32),
        ],
    )
    return k(sidx)


_BR = 1000


def _l2norm_body(x_ref, o_ref):
    x = x_ref[...]
    n = jnp.sqrt(jnp.sum(x * x, axis=1, keepdims=True))
    o_ref[...] = x / jnp.maximum(n, _EPS)


def _l2norm(x):
    return pl.pallas_call(
        _l2norm_body,
        grid=(N // _BR,),
        in_specs=[pl.BlockSpec((_BR, H), lambda i: (i, 0))],
        out_specs=pl.BlockSpec((_BR, H), lambda i: (i, 0)),
        out_shape=jax.ShapeDtypeStruct((N, H), jnp.float32),
    )(x)


def _relgru_body(s0_ref, s1_ref, c_ref, emb_ref,
                 wih_ref, whh_ref, bih_ref, bhh_ref, o_ref):
    cnt = jnp.maximum(c_ref[...][:NR2, 0:1], 1.0)
    x0 = s0_ref[...][:NR2] / cnt
    x1 = s1_ref[...][:NR2] / cnt
    emb = emb_ref[...]
    wih = wih_ref[...]
    gi = (jnp.dot(emb, wih[:H], preferred_element_type=jnp.float32)
          + jnp.dot(x0, wih[H:H + HW], preferred_element_type=jnp.float32)
          + jnp.dot(x1, wih[H + HW:], preferred_element_type=jnp.float32)
          + bih_ref[...])
    gh = jnp.dot(emb, whh_ref[...],
                 preferred_element_type=jnp.float32) + bhh_ref[...]
    r = jax.nn.sigmoid(gi[:, :H] + gh[:, :H])
    z = jax.nn.sigmoid(gi[:, H:2 * H] + gh[:, H:2 * H])
    n = jnp.tanh(gi[:, 2 * H:] + r * gh[:, 2 * H:])
    out = (1.0 - z) * n + z * emb
    nrm = jnp.sqrt(jnp.sum(out * out, axis=1, keepdims=True))
    o_ref[...] = out / jnp.maximum(nrm, _EPS)


def _relgru(s0, s1, cnt, emb_rel, wihT, whhT, bih, bhh):
    return pl.pallas_call(
        _relgru_body,
        grid=(1,),
        in_specs=[
            pl.BlockSpec((RR, HW), lambda k: (0, 0)),
            pl.BlockSpec((RR, HW), lambda k: (0, 0)),
            pl.BlockSpec((RR, HW), lambda k: (0, 0)),
            pl.BlockSpec((NR2, H), lambda k: (0, 0)),
            pl.BlockSpec((2 * H, 3 * H), lambda k: (0, 0)),
            pl.BlockSpec((H, 3 * H), lambda k: (0, 0)),
            pl.BlockSpec((1, 3 * H), lambda k: (0, 0)),
            pl.BlockSpec((1, 3 * H), lambda k: (0, 0)),
        ],
        out_specs=pl.BlockSpec((NR2, H), lambda k: (0, 0)),
        out_shape=jax.ShapeDtypeStruct((NR2, H), jnp.float32),
    )(s0, s1, cnt, emb_rel, wihT, whhT, bih, bhh)


def _layer_body(s0_ref, s1_ref, b0_ref, b1_ref, d0_ref, d1_ref,
                cur_ref, wn_ref, ws_ref, o_ref):
    deg = d0_ref[...][:, 0:1] + d1_ref[...][:, 0:1]
    inv = 1.0 / jnp.maximum(deg, 1.0)
    e0 = (s0_ref[...] + b0_ref[...]) * inv
    e1 = (s1_ref[...] + b1_ref[...]) * inv
    wn = wn_ref[...]
    y = (jnp.dot(e0, wn[:HW], preferred_element_type=jnp.float32)
         + jnp.dot(e1, wn[HW:], preferred_element_type=jnp.float32)
         + jnp.dot(cur_ref[...], ws_ref[...],
                   preferred_element_type=jnp.float32))
    o_ref[...] = jnp.where(y >= 0, y, _SLOPE * y)


def _layer(s0, s1, b0, b1, d0, d1, cur, W_n, W_s):
    rspec = pl.BlockSpec((_BR, HW), lambda i: (i, 0))
    return pl.pallas_call(
        _layer_body,
        grid=(N // _BR,),
        in_specs=[
            rspec, rspec, rspec, rspec, rspec, rspec,
            pl.BlockSpec((_BR, H), lambda i: (i, 0)),
            pl.BlockSpec((H, H), lambda i: (0, 0)),
            pl.BlockSpec((H, H), lambda i: (0, 0)),
        ],
        out_specs=pl.BlockSpec((_BR, H), lambda i: (i, 0)),
        out_shape=jax.ShapeDtypeStruct((N, H), jnp.float32),
    )(s0, s1, b0, b1, d0, d1, cur, W_n, W_s)


def _final_body(cur_ref, h_ref, wih_ref, whh_ref, bih_ref, bhh_ref,
                wt_ref, bt_ref, o_ref):
    cur = cur_ref[...]
    nrm = jnp.sqrt(jnp.sum(cur * cur, axis=1, keepdims=True))
    x = cur / jnp.maximum(nrm, _EPS)
    hb = h_ref[...]
    gi = jnp.dot(x, wih_ref[...],
                 preferred_element_type=jnp.float32) + bih_ref[...]
    gh = jnp.dot(hb, whh_ref[...],
                 preferred_element_type=jnp.float32) + bhh_ref[...]
    r = jax.nn.sigmoid(gi[:, :H] + gh[:, :H])
    z = jax.nn.sigmoid(gi[:, H:2 * H] + gh[:, H:2 * H])
    n = jnp.tanh(gi[:, 2 * H:] + r * gh[:, 2 * H:])
    hn = (1.0 - z) * n + z * hb
    nrm2 = jnp.sqrt(jnp.sum(hn * hn, axis=1, keepdims=True))
    hn = hn / jnp.maximum(nrm2, _EPS)
    gate = jax.nn.sigmoid(
        jnp.dot(hn, wt_ref[...], preferred_element_type=jnp.float32)
        + bt_ref[...])
    o_ref[...] = gate * hn + (1.0 - gate) * hb


def _final(cur, h, wihT, whhT, bih, bhh, wt, bt):
    return pl.pallas_call(
        _final_body,
        grid=(N // _BR,),
        in_specs=[
            pl.BlockSpec((_BR, H), lambda i: (i, 0)),
            pl.BlockSpec((_BR, H), lambda i: (i, 0)),
            pl.BlockSpec((H, 3 * H), lambda i: (0, 0)),
            pl.BlockSpec((H, 3 * H), lambda i: (0, 0)),
            pl.BlockSpec((1, 3 * H), lambda i: (0, 0)),
            pl.BlockSpec((1, 3 * H), lambda i: (0, 0)),
            pl.BlockSpec((H, H), lambda i: (0, 0)),
            pl.BlockSpec((1, H), lambda i: (0, 0)),
        ],
        out_specs=pl.BlockSpec((_BR, H), lambda i: (i, 0)),
        out_shape=jax.ShapeDtypeStruct((N, H), jnp.float32),
    )(cur, h, wihT, whhT, bih, bhh, wt, bt)


def _gidx(idx):
    g = jnp.pad(2 * idx, (0, EP - E))
    return jnp.stack([g, g + 1])


def kernel(edge_index, etype, r_to_e, rel_seg, dynamic_emb, emb_rel,
           W_ih_r, W_hh_r, b_ih_r, b_hh_r,
           W_ih_e, W_hh_e, b_ih_e, b_hh_e,
           W_n1, W_s1, W_n2, W_s2,
           time_gate_weight, time_gate_bias):
    srcv = edge_index[0]
    dstv = edge_index[1]
    dstp = jnp.pad(dstv, (0, EP - E), constant_values=N)
    segp = jnp.pad(rel_seg, (0, EP - E), constant_values=NR2)
    g_r2e = _gidx(r_to_e)
    g_et = _gidx(etype)
    g_src = _gidx(srcv)

    h = _l2norm(dynamic_emb)
    sums, cnts = _scat_rel(h.reshape(2 * N, HW), g_r2e, segp)
    h0 = _relgru(sums[0], sums[1], cnts[0], emb_rel,
                 W_ih_r.T, W_hh_r.T, b_ih_r[None, :], b_hh_r[None, :])

    d = _scat_deg(dstp)
    B = _scat_dst(h0.reshape(2 * NR2, HW), g_et, dstp)

    cur = h
    for W_n, W_s in ((W_n1, W_s1), (W_n2, W_s2)):
        S = _scat_dst(cur.reshape(2 * N, HW), g_src, dstp)
        cur = _layer(S[0], S[1], B[0], B[1], d[0], d[1], cur, W_n, W_s)

    return _final(cur, h, W_ih_e.T, W_hh_e.T,
                  b_ih_e[None, :], b_hh_e[None, :],
                  time_gate_weight, time_gate_bias[None, :])


# speedup vs baseline: 2.6982x; 1.2871x over previous
import functools

import jax
import jax.numpy as jnp
from jax import lax
from jax.experimental import pallas as pl
from jax.experimental.pallas import tpu as pltpu
from jax.experimental.pallas import tpu_sc as plsc

N = 10000
E = 160000
H = 256
NR2 = 200

HW = 128
NC = 2
NS = 16
CH = 128
EP = NC * NS * CH * 40
RD = 10112
RR = 256

_SLOPE = (1.0 / 8.0 + 1.0 / 3.0) / 2.0
_EPS = 1e-12

_mesh = plsc.VectorSubcoreMesh(core_axis_name="c", subcore_axis_name="s")


def _fill(buf, nrows, val):
    v = jnp.full((16,), val, jnp.float32)

    def fr(j, _):
        def fc(t, _):
            buf[j, pl.ds(t * 16, 16)] = v
            return 0
        lax.fori_loop(0, HW // 16, fc, 0)
        return 0
    lax.fori_loop(0, nrows, fr, 0)


def _zero_slab(slab, zbuf, s, rpt):
    full, rem = rpt // CH, rpt % CH

    def za(j, _):
        pltpu.sync_copy(zbuf, slab.at[pl.ds(s * rpt + j * CH, CH)])
        return 0
    if full:
        lax.fori_loop(0, full, za, 0)
    if rem:
        pltpu.sync_copy(zbuf.at[pl.ds(0, rem)],
                        slab.at[pl.ds(s * rpt + full * CH, rem)])


def _copy_out(slab, out, c, s, rpt):
    full, rem = rpt // CH, rpt % CH

    def ca(j, _):
        r = s * rpt + j * CH
        pltpu.sync_copy(slab.at[pl.ds(r, CH)], out.at[c, pl.ds(r, CH)])
        return 0
    if full:
        lax.fori_loop(0, full, ca, 0)
    if rem:
        r = s * rpt + full * CH
        pltpu.sync_copy(slab.at[pl.ds(r, rem)], out.at[c, pl.ds(r, rem)])


NCH = EP // NS // CH


def _rowscat_body(nrows, cnt, *refs):
    if cnt:
        (tbl, gidx, sidx, osum, ocnt,
         gi2, si2, rows0, rows1, obuf, acc, cacc, sem0, sem1) = refs
    else:
        (tbl, gidx, sidx, osum,
         gi2, si2, rows0, rows1, acc, sem0, sem1) = refs

    c = lax.axis_index("c")
    s = lax.axis_index("s")
    rpt = nrows // NS

    _fill(rows0, CH, 0.0)
    _zero_slab(acc, rows0, s, rpt)
    if cnt:
        _zero_slab(cacc, rows0, s, rpt)
        _fill(obuf, CH, 1.0)

    plsc.subcore_barrier()

    nh = NCH // 2

    def half(hf):
        pltpu.sync_copy(gidx.at[c, s, pl.ds(hf * nh, nh)], gi2)
        pltpu.sync_copy(sidx.at[s, pl.ds(hf * nh, nh)], si2)
        pltpu.async_copy(tbl.at[gi2.at[0]], rows0, sem0)

        def pair(j, _):
            i0 = 2 * j
            i1 = i0 + 1
            pltpu.make_async_copy(tbl.at[gi2.at[i0]], rows0, sem0).wait()
            pltpu.async_copy(tbl.at[gi2.at[i1]], rows1, sem1)
            pltpu.sync_copy(rows0, acc.at[si2.at[i0]], add=True)
            if cnt:
                @pl.when(c == 0)
                def _():
                    pltpu.sync_copy(obuf, cacc.at[si2.at[i0]], add=True)
            pltpu.make_async_copy(tbl.at[gi2.at[i1]], rows1, sem1).wait()

            @pl.when(j < nh // 2 - 1)
            def _():
                pltpu.async_copy(tbl.at[gi2.at[i0 + 2]], rows0, sem0)
            pltpu.sync_copy(rows1, acc.at[si2.at[i1]], add=True)
            if cnt:
                @pl.when(c == 0)
                def _():
                    pltpu.sync_copy(obuf, cacc.at[si2.at[i1]], add=True)
            return 0
        lax.fori_loop(0, nh // 2, pair, 0)

    half(0)
    half(1)

    plsc.subcore_barrier()

    _copy_out(acc, osum, c, s, rpt)
    if cnt:
        @pl.when(c == 0)
        def _():
            _copy_out(cacc, ocnt, 0, s, rpt)


def _make_rowscat(nrows, cnt):
    scratch = [
        pltpu.VMEM((NCH // 2, CH), jnp.int32),
        pltpu.VMEM((NCH // 2, CH), jnp.int32),
        pltpu.VMEM((CH, HW), jnp.float32),
        pltpu.VMEM((CH, HW), jnp.float32),
    ]
    if cnt:
        scratch.append(pltpu.VMEM((CH, HW), jnp.float32))
    scratch.append(pltpu.VMEM_SHARED((nrows, HW), jnp.float32))
    if cnt:
        scratch.append(pltpu.VMEM_SHARED((nrows, HW), jnp.float32))
    scratch.append(pltpu.SemaphoreType.DMA)
    scratch.append(pltpu.SemaphoreType.DMA)

    out_type = [jax.ShapeDtypeStruct((NC, nrows, HW), jnp.float32)]
    if cnt:
        out_type.append(jax.ShapeDtypeStruct((1, nrows, HW), jnp.float32))

    @jax.jit
    def run(tbl, gidx, sidx):
        k = pl.kernel(
            functools.partial(_rowscat_body, nrows, cnt),
            out_type=tuple(out_type) if cnt else out_type[0],
            mesh=_mesh,
            scratch_types=scratch,
        )
        return k(tbl, gidx, sidx)
    return run


_scat_rel = _make_rowscat(RR, True)
_scat_dst = _make_rowscat(RD, False)


def _deg_body(sidx, ocnt, si2, obuf, acc):
    c = lax.axis_index("c")
    s = lax.axis_index("s")
    rpt = RD // NS

    pltpu.sync_copy(sidx.at[c, s], si2)

    _fill(obuf, CH, 0.0)
    _zero_slab(acc, obuf, s, rpt)
    _fill(obuf, CH, 1.0)

    plsc.subcore_barrier()

    def chunk(i, _):
        pltpu.sync_copy(obuf, acc.at[si2.at[i]], add=True)
        return 0
    lax.fori_loop(0, NCH // NC, chunk, 0)

    plsc.subcore_barrier()
    _copy_out(acc, ocnt, c, s, rpt)


@jax.jit
def _scat_deg(sidx):
    k = pl.kernel(
        _deg_body,
        out_type=jax.ShapeDtypeStruct((NC, RD, HW), jnp.float32),
        mesh=_mesh,
        scratch_types=[
            pltpu.VMEM((NCH // NC, CH), jnp.int32),
            pltpu.VMEM((CH, HW), jnp.float32),
            pltpu.VMEM_SHARED((RD, HW), jnp.float32),
        ],
    )
    return k(sidx)


_BR = 1000


def _l2norm_body(x_ref, o_ref):
    x = x_ref[...]
    n = jnp.sqrt(jnp.sum(x * x, axis=1, keepdims=True))
    o_ref[...] = x / jnp.maximum(n, _EPS)


def _l2norm(x):
    return pl.pallas_call(
        _l2norm_body,
        grid=(N // _BR,),
        in_specs=[pl.BlockSpec((_BR, H), lambda i: (i, 0))],
        out_specs=pl.BlockSpec((_BR, H), lambda i: (i, 0)),
        out_shape=jax.ShapeDtypeStruct((N, H), jnp.float32),
    )(x)


def _relgru_body(s0_ref, s1_ref, c_ref, emb_ref,
                 wih_ref, whh_ref, bih_ref, bhh_ref, o_ref):
    cnt = jnp.maximum(c_ref[...][:NR2, 0:1], 1.0)
    x0 = s0_ref[...][:NR2] / cnt
    x1 = s1_ref[...][:NR2] / cnt
    emb = emb_ref[...]
    wih = wih_ref[...]
    gi = (jnp.dot(emb, wih[:H], preferred_element_type=jnp.float32)
          + jnp.dot(x0, wih[H:H + HW], preferred_element_type=jnp.float32)
          + jnp.dot(x1, wih[H + HW:], preferred_element_type=jnp.float32)
          + bih_ref[...])
    gh = jnp.dot(emb, whh_ref[...],
                 preferred_element_type=jnp.float32) + bhh_ref[...]
    r = jax.nn.sigmoid(gi[:, :H] + gh[:, :H])
    z = jax.nn.sigmoid(gi[:, H:2 * H] + gh[:, H:2 * H])
    n = jnp.tanh(gi[:, 2 * H:] + r * gh[:, 2 * H:])
    out = (1.0 - z) * n + z * emb
    nrm = jnp.sqrt(jnp.sum(out * out, axis=1, keepdims=True))
    o_ref[...] = out / jnp.maximum(nrm, _EPS)


def _relgru(s0, s1, cnt, emb_rel, wihT, whhT, bih, bhh):
    return pl.pallas_call(
        _relgru_body,
        grid=(1,),
        in_specs=[
            pl.BlockSpec((RR, HW), lambda k: (0, 0)),
            pl.BlockSpec((RR, HW), lambda k: (0, 0)),
            pl.BlockSpec((RR, HW), lambda k: (0, 0)),
            pl.BlockSpec((NR2, H), lambda k: (0, 0)),
            pl.BlockSpec((2 * H, 3 * H), lambda k: (0, 0)),
            pl.BlockSpec((H, 3 * H), lambda k: (0, 0)),
            pl.BlockSpec((1, 3 * H), lambda k: (0, 0)),
            pl.BlockSpec((1, 3 * H), lambda k: (0, 0)),
        ],
        out_specs=pl.BlockSpec((NR2, H), lambda k: (0, 0)),
        out_shape=jax.ShapeDtypeStruct((NR2, H), jnp.float32),
    )(s0, s1, cnt, emb_rel, wihT, whhT, bih, bhh)


def _layer_body(s0_ref, s1_ref, b0_ref, b1_ref, d0_ref, d1_ref,
                cur_ref, wn_ref, ws_ref, o_ref):
    deg = d0_ref[...][:, 0:1] + d1_ref[...][:, 0:1]
    inv = 1.0 / jnp.maximum(deg, 1.0)
    e0 = (s0_ref[...] + b0_ref[...]) * inv
    e1 = (s1_ref[...] + b1_ref[...]) * inv
    wn = wn_ref[...]
    y = (jnp.dot(e0, wn[:HW], preferred_element_type=jnp.float32)
         + jnp.dot(e1, wn[HW:], preferred_element_type=jnp.float32)
         + jnp.dot(cur_ref[...], ws_ref[...],
                   preferred_element_type=jnp.float32))
    o_ref[...] = jnp.where(y >= 0, y, _SLOPE * y)


def _layer(s0, s1, b0, b1, d0, d1, cur, W_n, W_s):
    rspec = pl.BlockSpec((_BR, HW), lambda i: (i, 0))
    return pl.pallas_call(
        _layer_body,
        grid=(N // _BR,),
        in_specs=[
            rspec, rspec, rspec, rspec, rspec, rspec,
            pl.BlockSpec((_BR, H), lambda i: (i, 0)),
            pl.BlockSpec((H, H), lambda i: (0, 0)),
            pl.BlockSpec((H, H), lambda i: (0, 0)),
        ],
        out_specs=pl.BlockSpec((_BR, H), lambda i: (i, 0)),
        out_shape=jax.ShapeDtypeStruct((N, H), jnp.float32),
    )(s0, s1, b0, b1, d0, d1, cur, W_n, W_s)


def _final_body(cur_ref, h_ref, wih_ref, whh_ref, bih_ref, bhh_ref,
                wt_ref, bt_ref, o_ref):
    cur = cur_ref[...]
    nrm = jnp.sqrt(jnp.sum(cur * cur, axis=1, keepdims=True))
    x = cur / jnp.maximum(nrm, _EPS)
    hb = h_ref[...]
    gi = jnp.dot(x, wih_ref[...],
                 preferred_element_type=jnp.float32) + bih_ref[...]
    gh = jnp.dot(hb, whh_ref[...],
                 preferred_element_type=jnp.float32) + bhh_ref[...]
    r = jax.nn.sigmoid(gi[:, :H] + gh[:, :H])
    z = jax.nn.sigmoid(gi[:, H:2 * H] + gh[:, H:2 * H])
    n = jnp.tanh(gi[:, 2 * H:] + r * gh[:, 2 * H:])
    hn = (1.0 - z) * n + z * hb
    nrm2 = jnp.sqrt(jnp.sum(hn * hn, axis=1, keepdims=True))
    hn = hn / jnp.maximum(nrm2, _EPS)
    gate = jax.nn.sigmoid(
        jnp.dot(hn, wt_ref[...], preferred_element_type=jnp.float32)
        + bt_ref[...])
    o_ref[...] = gate * hn + (1.0 - gate) * hb


def _final(cur, h, wihT, whhT, bih, bhh, wt, bt):
    return pl.pallas_call(
        _final_body,
        grid=(N // _BR,),
        in_specs=[
            pl.BlockSpec((_BR, H), lambda i: (i, 0)),
            pl.BlockSpec((_BR, H), lambda i: (i, 0)),
            pl.BlockSpec((H, 3 * H), lambda i: (0, 0)),
            pl.BlockSpec((H, 3 * H), lambda i: (0, 0)),
            pl.BlockSpec((1, 3 * H), lambda i: (0, 0)),
            pl.BlockSpec((1, 3 * H), lambda i: (0, 0)),
            pl.BlockSpec((H, H), lambda i: (0, 0)),
            pl.BlockSpec((1, H), lambda i: (0, 0)),
        ],
        out_specs=pl.BlockSpec((_BR, H), lambda i: (i, 0)),
        out_shape=jax.ShapeDtypeStruct((N, H), jnp.float32),
    )(cur, h, wihT, whhT, bih, bhh, wt, bt)


def _gidx(idx):
    g = jnp.pad(2 * idx, (0, EP - E))
    return jnp.stack([g, g + 1]).reshape(NC, NS, NCH, CH)


def kernel(edge_index, etype, r_to_e, rel_seg, dynamic_emb, emb_rel,
           W_ih_r, W_hh_r, b_ih_r, b_hh_r,
           W_ih_e, W_hh_e, b_ih_e, b_hh_e,
           W_n1, W_s1, W_n2, W_s2,
           time_gate_weight, time_gate_bias):
    srcv = edge_index[0]
    dstv = edge_index[1]
    dstp = jnp.pad(dstv, (0, EP - E), constant_values=N)
    dst_t = dstp.reshape(NS, NCH, CH)
    dst_c = dstp.reshape(NC, NS, NCH // NC, CH)
    segp = jnp.pad(rel_seg, (0, EP - E),
                   constant_values=NR2).reshape(NS, NCH, CH)
    g_r2e = _gidx(r_to_e)
    g_et = _gidx(etype)
    g_src = _gidx(srcv)

    h = _l2norm(dynamic_emb)
    sums, cnts = _scat_rel(h.reshape(2 * N, HW), g_r2e, segp)
    h0 = _relgru(sums[0], sums[1], cnts[0], emb_rel,
                 W_ih_r.T, W_hh_r.T, b_ih_r[None, :], b_hh_r[None, :])

    d = _scat_deg(dst_c)
    B = _scat_dst(h0.reshape(2 * NR2, HW), g_et, dst_t)

    cur = h
    for W_n, W_s in ((W_n1, W_s1), (W_n2, W_s2)):
        S = _scat_dst(cur.reshape(2 * N, HW), g_src, dst_t)
        cur = _layer(S[0], S[1], B[0], B[1], d[0], d[1], cur, W_n, W_s)

    return _final(cur, h, W_ih_e.T, W_hh_e.T,
                  b_ih_e[None, :], b_hh_e[None, :],
                  time_gate_weight, time_gate_bias[None, :])


# speedup vs baseline: 2.8211x; 1.0455x over previous
import functools

import jax
import jax.numpy as jnp
from jax import lax
from jax.experimental import pallas as pl
from jax.experimental.pallas import tpu as pltpu
from jax.experimental.pallas import tpu_sc as plsc

N = 10000
E = 160000
H = 256
NR2 = 200

HW = 128
NC = 2
NS = 16
CH = 128
EP = NC * NS * CH * 40
NCH = EP // NS // CH
RD = 10112
RDS = RD + 256

_SLOPE = (1.0 / 8.0 + 1.0 / 3.0) / 2.0
_EPS = 1e-12

_mesh = plsc.VectorSubcoreMesh(core_axis_name="c", subcore_axis_name="s")


def _fill(buf, nrows, width, val):
    v = jnp.full((16,), val, jnp.float32)

    def fr(j, _):
        for t in range(width // 16):
            buf[j, pl.ds(t * 16, 16)] = v
        return 0
    lax.fori_loop(0, nrows, fr, 0)


def _fill_zero(buf, nrows, width):
    _fill(buf, nrows, width, 0.0)


def _fill_one(buf, nrows, width):
    _fill(buf, nrows, width, 1.0)


def _zero_slab(slab, zbuf, s, rpt):
    full, rem = rpt // CH, rpt % CH

    def za(j, _):
        pltpu.sync_copy(zbuf, slab.at[pl.ds(s * rpt + j * CH, CH)])
        return 0
    if full:
        lax.fori_loop(0, full, za, 0)
    if rem:
        pltpu.sync_copy(zbuf.at[pl.ds(0, rem)],
                        slab.at[pl.ds(s * rpt + full * CH, rem)])


def _copy_out(slab, out, c, s, rpt):
    full, rem = rpt // CH, rpt % CH

    def ca(j, _):
        r = s * rpt + j * CH
        pltpu.sync_copy(slab.at[pl.ds(r, CH)], out.at[c, pl.ds(r, CH)])
        return 0
    if full:
        lax.fori_loop(0, full, ca, 0)
    if rem:
        r = s * rpt + full * CH
        pltpu.sync_copy(slab.at[pl.ds(r, rem)], out.at[c, pl.ds(r, rem)])


def _rowscat_body(nrows, width, nch, nb, *refs):
    (tbl, gidx, sidx, osum,
     gi2, si2, rows0, rows1, acc, sem0, sem1) = refs

    c = lax.axis_index("c")
    s = lax.axis_index("s")
    rpt = nrows // NS

    _fill_zero(rows0, CH, width)
    _zero_slab(acc, rows0, s, rpt)

    plsc.subcore_barrier()

    def block(l):
        pltpu.sync_copy(gidx.at[c, s, pl.ds(l * nb, nb)], gi2)
        pltpu.sync_copy(sidx.at[s, pl.ds(l * nb, nb)], si2)
        pltpu.async_copy(tbl.at[gi2.at[0]], rows0, sem0)

        def pair(j, _):
            i0 = 2 * j
            i1 = i0 + 1
            pltpu.make_async_copy(tbl.at[gi2.at[i0]], rows0, sem0).wait()
            pltpu.async_copy(tbl.at[gi2.at[i1]], rows1, sem1)
            pltpu.sync_copy(rows0, acc.at[si2.at[i0]], add=True)
            pltpu.make_async_copy(tbl.at[gi2.at[i1]], rows1, sem1).wait()

            @pl.when(j < nb // 2 - 1)
            def _():
                pltpu.async_copy(tbl.at[gi2.at[i0 + 2]], rows0, sem0)
            pltpu.sync_copy(rows1, acc.at[si2.at[i1]], add=True)
            return 0
        lax.fori_loop(0, nb // 2, pair, 0)

    for l in range(nch // nb):
        block(l)

    plsc.subcore_barrier()
    _copy_out(acc, osum, c, s, rpt)


def _make_rowscat(nrows, width, nch, nb):
    scratch = [
        pltpu.VMEM((nb, CH), jnp.int32),
        pltpu.VMEM((nb, CH), jnp.int32),
        pltpu.VMEM((CH, width), jnp.float32),
        pltpu.VMEM((CH, width), jnp.float32),
        pltpu.VMEM_SHARED((nrows, width), jnp.float32),
        pltpu.SemaphoreType.DMA,
        pltpu.SemaphoreType.DMA,
    ]

    @jax.jit
    def run(tbl, gidx, sidx):
        k = pl.kernel(
            functools.partial(_rowscat_body, nrows, width, nch, nb),
            out_type=jax.ShapeDtypeStruct((NC, nrows, width), jnp.float32),
            mesh=_mesh,
            scratch_types=scratch,
        )
        return k(tbl, gidx, sidx)
    return run


_scat_merged = _make_rowscat(RDS, HW, 2 * NCH, 32)
_scat_dst = _make_rowscat(RD, HW, NCH, 40)


def _cnt_body(sidx, ocnt, si2, obuf, acc):
    c = lax.axis_index("c")
    s = lax.axis_index("s")
    rpt = RDS // NS

    pltpu.sync_copy(sidx.at[c, s], si2)

    _fill_zero(obuf, CH, HW)
    _zero_slab(acc, obuf, s, rpt)
    _fill_one(obuf, CH, HW)

    plsc.subcore_barrier()

    def chunk(i, _):
        pltpu.sync_copy(obuf, acc.at[si2.at[i]], add=True)
        return 0
    lax.fori_loop(0, NCH, chunk, 0)

    plsc.subcore_barrier()
    _copy_out(acc, ocnt, c, s, rpt)


@jax.jit
def _scat_cnt(sidx):
    k = pl.kernel(
        _cnt_body,
        out_type=jax.ShapeDtypeStruct((NC, RDS, HW), jnp.float32),
        mesh=_mesh,
        scratch_types=[
            pltpu.VMEM((NCH, CH), jnp.int32),
            pltpu.VMEM((CH, HW), jnp.float32),
            pltpu.VMEM_SHARED((RDS, HW), jnp.float32),
        ],
    )
    return k(sidx)


_BR = 1000


def _l2norm_body(x_ref, o_ref):
    x = x_ref[...]
    n = jnp.sqrt(jnp.sum(x * x, axis=1, keepdims=True))
    o_ref[...] = x / jnp.maximum(n, _EPS)


def _l2norm(x):
    return pl.pallas_call(
        _l2norm_body,
        grid=(N // _BR,),
        in_specs=[pl.BlockSpec((_BR, H), lambda i: (i, 0))],
        out_specs=pl.BlockSpec((_BR, H), lambda i: (i, 0)),
        out_shape=jax.ShapeDtypeStruct((N, H), jnp.float32),
    )(x)


def _relgru_body(s0_ref, s1_ref, c0_ref, c1_ref, emb_ref,
                 wih_ref, whh_ref, bih_ref, bhh_ref, o_ref):
    s0 = s0_ref[...]
    s1 = s1_ref[...]
    cnt = jnp.maximum(c0_ref[...][:, 0:1] + c1_ref[...][:, 0:1], 1.0)
    x0 = s0 / cnt
    x1 = s1 / cnt
    emb = emb_ref[...]
    wih = wih_ref[...]
    gi = (jnp.dot(emb, wih[:H], preferred_element_type=jnp.float32)
          + jnp.dot(x0, wih[H:H + HW], preferred_element_type=jnp.float32)
          + jnp.dot(x1, wih[H + HW:], preferred_element_type=jnp.float32)
          + bih_ref[...])
    gh = jnp.dot(emb, whh_ref[...],
                 preferred_element_type=jnp.float32) + bhh_ref[...]
    r = jax.nn.sigmoid(gi[:, :H] + gh[:, :H])
    z = jax.nn.sigmoid(gi[:, H:2 * H] + gh[:, H:2 * H])
    n = jnp.tanh(gi[:, 2 * H:] + r * gh[:, 2 * H:])
    out = (1.0 - z) * n + z * emb
    nrm = jnp.sqrt(jnp.sum(out * out, axis=1, keepdims=True))
    o_ref[...] = out / jnp.maximum(nrm, _EPS)


def _relgru(s0, s1, c0, c1, emb_rel, wihT, whhT, bih, bhh):
    return pl.pallas_call(
        _relgru_body,
        grid=(1,),
        in_specs=[
            pl.BlockSpec((NR2, HW), lambda k: (0, 0)),
            pl.BlockSpec((NR2, HW), lambda k: (0, 0)),
            pl.BlockSpec((NR2, HW), lambda k: (0, 0)),
            pl.BlockSpec((NR2, HW), lambda k: (0, 0)),
            pl.BlockSpec((NR2, H), lambda k: (0, 0)),
            pl.BlockSpec((2 * H, 3 * H), lambda k: (0, 0)),
            pl.BlockSpec((H, 3 * H), lambda k: (0, 0)),
            pl.BlockSpec((1, 3 * H), lambda k: (0, 0)),
            pl.BlockSpec((1, 3 * H), lambda k: (0, 0)),
        ],
        out_specs=pl.BlockSpec((NR2, H), lambda k: (0, 0)),
        out_shape=jax.ShapeDtypeStruct((NR2, H), jnp.float32),
    )(s0, s1, c0, c1, emb_rel, wihT, whhT, bih, bhh)


def _layer_body(s0_ref, s1_ref, b0_ref, b1_ref, d0_ref, d1_ref,
                cur_ref, wn_ref, ws_ref, o_ref):
    deg = d0_ref[...][:, 0:1] + d1_ref[...][:, 0:1]
    inv = 1.0 / jnp.maximum(deg, 1.0)
    e0 = (s0_ref[...] + b0_ref[...]) * inv
    e1 = (s1_ref[...] + b1_ref[...]) * inv
    wn = wn_ref[...]
    y = (jnp.dot(e0, wn[:HW], preferred_element_type=jnp.float32)
         + jnp.dot(e1, wn[HW:], preferred_element_type=jnp.float32)
         + jnp.dot(cur_ref[...], ws_ref[...],
                   preferred_element_type=jnp.float32))
    o_ref[...] = jnp.where(y >= 0, y, _SLOPE * y)


def _layer(s0, s1, b0, b1, d0, d1, cur, W_n, W_s):
    rspec = pl.BlockSpec((_BR, HW), lambda i: (i, 0))
    return pl.pallas_call(
        _layer_body,
        grid=(N // _BR,),
        in_specs=[
            rspec, rspec, rspec, rspec, rspec, rspec,
            pl.BlockSpec((_BR, H), lambda i: (i, 0)),
            pl.BlockSpec((H, H), lambda i: (0, 0)),
            pl.BlockSpec((H, H), lambda i: (0, 0)),
        ],
        out_specs=pl.BlockSpec((_BR, H), lambda i: (i, 0)),
        out_shape=jax.ShapeDtypeStruct((N, H), jnp.float32),
    )(s0, s1, b0, b1, d0, d1, cur, W_n, W_s)


def _final_body(cur_ref, h_ref, wih_ref, whh_ref, bih_ref, bhh_ref,
                wt_ref, bt_ref, o_ref):
    cur = cur_ref[...]
    nrm = jnp.sqrt(jnp.sum(cur * cur, axis=1, keepdims=True))
    x = cur / jnp.maximum(nrm, _EPS)
    hb = h_ref[...]
    gi = jnp.dot(x, wih_ref[...],
                 preferred_element_type=jnp.float32) + bih_ref[...]
    gh = jnp.dot(hb, whh_ref[...],
                 preferred_element_type=jnp.float32) + bhh_ref[...]
    r = jax.nn.sigmoid(gi[:, :H] + gh[:, :H])
    z = jax.nn.sigmoid(gi[:, H:2 * H] + gh[:, H:2 * H])
    n = jnp.tanh(gi[:, 2 * H:] + r * gh[:, 2 * H:])
    hn = (1.0 - z) * n + z * hb
    nrm2 = jnp.sqrt(jnp.sum(hn * hn, axis=1, keepdims=True))
    hn = hn / jnp.maximum(nrm2, _EPS)
    gate = jax.nn.sigmoid(
        jnp.dot(hn, wt_ref[...], preferred_element_type=jnp.float32)
        + bt_ref[...])
    o_ref[...] = gate * hn + (1.0 - gate) * hb


def _final(cur, h, wihT, whhT, bih, bhh, wt, bt):
    return pl.pallas_call(
        _final_body,
        grid=(N // _BR,),
        in_specs=[
            pl.BlockSpec((_BR, H), lambda i: (i, 0)),
            pl.BlockSpec((_BR, H), lambda i: (i, 0)),
            pl.BlockSpec((H, 3 * H), lambda i: (0, 0)),
            pl.BlockSpec((H, 3 * H), lambda i: (0, 0)),
            pl.BlockSpec((1, 3 * H), lambda i: (0, 0)),
            pl.BlockSpec((1, 3 * H), lambda i: (0, 0)),
            pl.BlockSpec((H, H), lambda i: (0, 0)),
            pl.BlockSpec((1, H), lambda i: (0, 0)),
        ],
        out_specs=pl.BlockSpec((_BR, H), lambda i: (i, 0)),
        out_shape=jax.ShapeDtypeStruct((N, H), jnp.float32),
    )(cur, h, wihT, whhT, bih, bhh, wt, bt)


def _gidx1(idx):
    g = jnp.pad(2 * idx, (0, EP - E))
    return jnp.stack([g, g + 1])


def kernel(edge_index, etype, r_to_e, rel_seg, dynamic_emb, emb_rel,
           W_ih_r, W_hh_r, b_ih_r, b_hh_r,
           W_ih_e, W_hh_e, b_ih_e, b_hh_e,
           W_n1, W_s1, W_n2, W_s2,
           time_gate_weight, time_gate_bias):
    srcv = edge_index[0]
    dstv = edge_index[1]
    dstp = jnp.pad(dstv, (0, EP - E), constant_values=N)
    segp = jnp.pad(rel_seg, (0, EP - E), constant_values=NR2) + RD
    s_cat = jnp.concatenate([dstp, segp])
    g_m = jnp.concatenate([_gidx1(srcv), _gidx1(r_to_e)],
                          axis=1).reshape(NC, NS, 2 * NCH, CH)
    s_m = s_cat.reshape(NS, 2 * NCH, CH)
    s_c = s_cat.reshape(NC, NS, NCH, CH)
    g_et = _gidx1(etype).reshape(NC, NS, NCH, CH)
    g_src = _gidx1(srcv).reshape(NC, NS, NCH, CH)
    dst_t = dstp.reshape(NS, NCH, CH)

    h = _l2norm(dynamic_emb)

    C = _scat_cnt(s_c)
    M = _scat_merged(h.reshape(2 * N, HW), g_m, s_m)
    h0 = _relgru(M[0, RD:RD + NR2], M[1, RD:RD + NR2],
                 C[0, RD:RD + NR2], C[1, RD:RD + NR2], emb_rel,
                 W_ih_r.T, W_hh_r.T, b_ih_r[None, :], b_hh_r[None, :])

    B = _scat_dst(h0.reshape(2 * NR2, HW), g_et, dst_t)

    cur = _layer(M[0], M[1], B[0], B[1], C[0], C[1], h, W_n1, W_s1)
    S = _scat_dst(cur.reshape(2 * N, HW), g_src, dst_t)
    cur = _layer(S[0], S[1], B[0], B[1], C[0], C[1], cur, W_n2, W_s2)

    return _final(cur, h, W_ih_e.T, W_hh_e.T,
                  b_ih_e[None, :], b_hh_e[None, :],
                  time_gate_weight, time_gate_bias[None, :])


# speedup vs baseline: 2.8468x; 1.0091x over previous
import functools

import jax
import jax.numpy as jnp
from jax import lax
from jax.experimental import pallas as pl
from jax.experimental.pallas import tpu as pltpu
from jax.experimental.pallas import tpu_sc as plsc

N = 10000
E = 160000
H = 256
NR2 = 200

HW = 128
NC = 2
NS = 16
CH = 64
EP = NC * NS * CH * 80
NCH = EP // NS // CH
RD = 10112
RDS = RD + 256

_SLOPE = (1.0 / 8.0 + 1.0 / 3.0) / 2.0
_EPS = 1e-12

_mesh = plsc.VectorSubcoreMesh(core_axis_name="c", subcore_axis_name="s")


def _fill(buf, nrows, width, val):
    v = jnp.full((16,), val, jnp.float32)

    def fr(j, _):
        for t in range(width // 16):
            buf[j, pl.ds(t * 16, 16)] = v
        return 0
    lax.fori_loop(0, nrows, fr, 0)


def _fill_zero(buf, nrows, width):
    _fill(buf, nrows, width, 0.0)


def _fill_one(buf, nrows, width):
    _fill(buf, nrows, width, 1.0)


def _zero_slab(slab, zbuf, s, rpt):
    full, rem = rpt // CH, rpt % CH

    def za(j, _):
        pltpu.sync_copy(zbuf, slab.at[pl.ds(s * rpt + j * CH, CH)])
        return 0
    if full:
        lax.fori_loop(0, full, za, 0)
    if rem:
        pltpu.sync_copy(zbuf.at[pl.ds(0, rem)],
                        slab.at[pl.ds(s * rpt + full * CH, rem)])


def _copy_out(slab, out, c, s, rpt):
    full, rem = rpt // CH, rpt % CH

    def ca(j, _):
        r = s * rpt + j * CH
        pltpu.sync_copy(slab.at[pl.ds(r, CH)], out.at[c, pl.ds(r, CH)])
        return 0
    if full:
        lax.fori_loop(0, full, ca, 0)
    if rem:
        r = s * rpt + full * CH
        pltpu.sync_copy(slab.at[pl.ds(r, rem)], out.at[c, pl.ds(r, rem)])


def _rowscat_body(nrows, nch, nb, *refs):
    (tbl, gidx, sidx, osum,
     gi2, si2, b0, b1, b2, b3, acc, m0, m1, m2, m3) = refs
    bufs = (b0, b1, b2, b3)
    sems = (m0, m1, m2, m3)

    c = lax.axis_index("c")
    s = lax.axis_index("s")
    rpt = nrows // NS

    _fill_zero(b0, CH, HW)
    _zero_slab(acc, b0, s, rpt)

    plsc.subcore_barrier()

    def block(l):
        pltpu.sync_copy(gidx.at[c, s, pl.ds(l * nb, nb)], gi2)
        pltpu.sync_copy(sidx.at[s, pl.ds(l * nb, nb)], si2)
        for k in range(3):
            pltpu.async_copy(tbl.at[gi2.at[k]], bufs[k], sems[k])

        def quad(j, _):
            for k in range(4):
                i = 4 * j + k
                pltpu.make_async_copy(tbl.at[gi2.at[i]],
                                      bufs[k], sems[k]).wait()

                @pl.when(i + 3 < nb)
                def _(i=i, k=k):
                    kn = (k + 3) % 4
                    pltpu.async_copy(tbl.at[gi2.at[i + 3]],
                                     bufs[kn], sems[kn])
                pltpu.sync_copy(bufs[k], acc.at[si2.at[i]], add=True)
            return 0
        lax.fori_loop(0, nb // 4, quad, 0)

    for l in range(nch // nb):
        block(l)

    plsc.subcore_barrier()
    _copy_out(acc, osum, c, s, rpt)


def _make_rowscat(nrows, nch, nb):
    scratch = [
        pltpu.VMEM((nb, CH), jnp.int32),
        pltpu.VMEM((nb, CH), jnp.int32),
        pltpu.VMEM((CH, HW), jnp.float32),
        pltpu.VMEM((CH, HW), jnp.float32),
        pltpu.VMEM((CH, HW), jnp.float32),
        pltpu.VMEM((CH, HW), jnp.float32),
        pltpu.VMEM_SHARED((nrows, HW), jnp.float32),
        pltpu.SemaphoreType.DMA,
        pltpu.SemaphoreType.DMA,
        pltpu.SemaphoreType.DMA,
        pltpu.SemaphoreType.DMA,
    ]

    @jax.jit
    def run(tbl, gidx, sidx):
        k = pl.kernel(
            functools.partial(_rowscat_body, nrows, nch, nb),
            out_type=jax.ShapeDtypeStruct((NC, nrows, HW), jnp.float32),
            mesh=_mesh,
            scratch_types=scratch,
        )
        return k(tbl, gidx, sidx)
    return run


_scat_merged = _make_rowscat(RDS, 2 * NCH, 40)
_scat_dst = _make_rowscat(RD, NCH, 40)


def _cnt_body(sidx, ocnt, si2, obuf, acc):
    c = lax.axis_index("c")
    s = lax.axis_index("s")
    rpt = RDS // NS

    pltpu.sync_copy(sidx.at[c, s], si2)

    _fill_zero(obuf, CH, HW)
    _zero_slab(acc, obuf, s, rpt)
    _fill_one(obuf, CH, HW)

    plsc.subcore_barrier()

    def chunk(i, _):
        pltpu.sync_copy(obuf, acc.at[si2.at[i]], add=True)
        return 0
    lax.fori_loop(0, NCH, chunk, 0)

    plsc.subcore_barrier()
    _copy_out(acc, ocnt, c, s, rpt)


@jax.jit
def _scat_cnt(sidx):
    k = pl.kernel(
        _cnt_body,
        out_type=jax.ShapeDtypeStruct((NC, RDS, HW), jnp.float32),
        mesh=_mesh,
        scratch_types=[
            pltpu.VMEM((NCH, CH), jnp.int32),
            pltpu.VMEM((CH, HW), jnp.float32),
            pltpu.VMEM_SHARED((RDS, HW), jnp.float32),
        ],
    )
    return k(sidx)


_BR = 1000


def _l2norm_body(x_ref, o_ref):
    x = x_ref[...]
    n = jnp.sqrt(jnp.sum(x * x, axis=1, keepdims=True))
    o_ref[...] = x / jnp.maximum(n, _EPS)


def _l2norm(x):
    return pl.pallas_call(
        _l2norm_body,
        grid=(N // _BR,),
        in_specs=[pl.BlockSpec((_BR, H), lambda i: (i, 0))],
        out_specs=pl.BlockSpec((_BR, H), lambda i: (i, 0)),
        out_shape=jax.ShapeDtypeStruct((N, H), jnp.float32),
    )(x)


def _relgru_body(s0_ref, s1_ref, c0_ref, c1_ref, emb_ref,
                 wih_ref, whh_ref, bih_ref, bhh_ref, o_ref):
    s0 = s0_ref[...]
    s1 = s1_ref[...]
    cnt = jnp.maximum(c0_ref[...][:, 0:1] + c1_ref[...][:, 0:1], 1.0)
    x0 = s0 / cnt
    x1 = s1 / cnt
    emb = emb_ref[...]
    wih = wih_ref[...]
    gi = (jnp.dot(emb, wih[:H], preferred_element_type=jnp.float32)
          + jnp.dot(x0, wih[H:H + HW], preferred_element_type=jnp.float32)
          + jnp.dot(x1, wih[H + HW:], preferred_element_type=jnp.float32)
          + bih_ref[...])
    gh = jnp.dot(emb, whh_ref[...],
                 preferred_element_type=jnp.float32) + bhh_ref[...]
    r = jax.nn.sigmoid(gi[:, :H] + gh[:, :H])
    z = jax.nn.sigmoid(gi[:, H:2 * H] + gh[:, H:2 * H])
    n = jnp.tanh(gi[:, 2 * H:] + r * gh[:, 2 * H:])
    out = (1.0 - z) * n + z * emb
    nrm = jnp.sqrt(jnp.sum(out * out, axis=1, keepdims=True))
    o_ref[...] = out / jnp.maximum(nrm, _EPS)


def _relgru(s0, s1, c0, c1, emb_rel, wihT, whhT, bih, bhh):
    return pl.pallas_call(
        _relgru_body,
        grid=(1,),
        in_specs=[
            pl.BlockSpec((NR2, HW), lambda k: (0, 0)),
            pl.BlockSpec((NR2, HW), lambda k: (0, 0)),
            pl.BlockSpec((NR2, HW), lambda k: (0, 0)),
            pl.BlockSpec((NR2, HW), lambda k: (0, 0)),
            pl.BlockSpec((NR2, H), lambda k: (0, 0)),
            pl.BlockSpec((2 * H, 3 * H), lambda k: (0, 0)),
            pl.BlockSpec((H, 3 * H), lambda k: (0, 0)),
            pl.BlockSpec((1, 3 * H), lambda k: (0, 0)),
            pl.BlockSpec((1, 3 * H), lambda k: (0, 0)),
        ],
        out_specs=pl.BlockSpec((NR2, H), lambda k: (0, 0)),
        out_shape=jax.ShapeDtypeStruct((NR2, H), jnp.float32),
    )(s0, s1, c0, c1, emb_rel, wihT, whhT, bih, bhh)


def _layer_body(s0_ref, s1_ref, b0_ref, b1_ref, d0_ref, d1_ref,
                cur_ref, wn_ref, ws_ref, o_ref):
    deg = d0_ref[...][:, 0:1] + d1_ref[...][:, 0:1]
    inv = 1.0 / jnp.maximum(deg, 1.0)
    e0 = (s0_ref[...] + b0_ref[...]) * inv
    e1 = (s1_ref[...] + b1_ref[...]) * inv
    wn = wn_ref[...]
    y = (jnp.dot(e0, wn[:HW], preferred_element_type=jnp.float32)
         + jnp.dot(e1, wn[HW:], preferred_element_type=jnp.float32)
         + jnp.dot(cur_ref[...], ws_ref[...],
                   preferred_element_type=jnp.float32))
    o_ref[...] = jnp.where(y >= 0, y, _SLOPE * y)


def _layer(s0, s1, b0, b1, d0, d1, cur, W_n, W_s):
    rspec = pl.BlockSpec((_BR, HW), lambda i: (i, 0))
    return pl.pallas_call(
        _layer_body,
        grid=(N // _BR,),
        in_specs=[
            rspec, rspec, rspec, rspec, rspec, rspec,
            pl.BlockSpec((_BR, H), lambda i: (i, 0)),
            pl.BlockSpec((H, H), lambda i: (0, 0)),
            pl.BlockSpec((H, H), lambda i: (0, 0)),
        ],
        out_specs=pl.BlockSpec((_BR, H), lambda i: (i, 0)),
        out_shape=jax.ShapeDtypeStruct((N, H), jnp.float32),
    )(s0, s1, b0, b1, d0, d1, cur, W_n, W_s)


def _final_body(cur_ref, h_ref, wih_ref, whh_ref, bih_ref, bhh_ref,
                wt_ref, bt_ref, o_ref):
    cur = cur_ref[...]
    nrm = jnp.sqrt(jnp.sum(cur * cur, axis=1, keepdims=True))
    x = cur / jnp.maximum(nrm, _EPS)
    hb = h_ref[...]
    gi = jnp.dot(x, wih_ref[...],
                 preferred_element_type=jnp.float32) + bih_ref[...]
    gh = jnp.dot(hb, whh_ref[...],
                 preferred_element_type=jnp.float32) + bhh_ref[...]
    r = jax.nn.sigmoid(gi[:, :H] + gh[:, :H])
    z = jax.nn.sigmoid(gi[:, H:2 * H] + gh[:, H:2 * H])
    n = jnp.tanh(gi[:, 2 * H:] + r * gh[:, 2 * H:])
    hn = (1.0 - z) * n + z * hb
    nrm2 = jnp.sqrt(jnp.sum(hn * hn, axis=1, keepdims=True))
    hn = hn / jnp.maximum(nrm2, _EPS)
    gate = jax.nn.sigmoid(
        jnp.dot(hn, wt_ref[...], preferred_element_type=jnp.float32)
        + bt_ref[...])
    o_ref[...] = gate * hn + (1.0 - gate) * hb


def _final(cur, h, wihT, whhT, bih, bhh, wt, bt):
    return pl.pallas_call(
        _final_body,
        grid=(N // _BR,),
        in_specs=[
            pl.BlockSpec((_BR, H), lambda i: (i, 0)),
            pl.BlockSpec((_BR, H), lambda i: (i, 0)),
            pl.BlockSpec((H, 3 * H), lambda i: (0, 0)),
            pl.BlockSpec((H, 3 * H), lambda i: (0, 0)),
            pl.BlockSpec((1, 3 * H), lambda i: (0, 0)),
            pl.BlockSpec((1, 3 * H), lambda i: (0, 0)),
            pl.BlockSpec((H, H), lambda i: (0, 0)),
            pl.BlockSpec((1, H), lambda i: (0, 0)),
        ],
        out_specs=pl.BlockSpec((_BR, H), lambda i: (i, 0)),
        out_shape=jax.ShapeDtypeStruct((N, H), jnp.float32),
    )(cur, h, wihT, whhT, bih, bhh, wt, bt)


def _gidx1(idx):
    g = jnp.pad(2 * idx, (0, EP - E))
    return jnp.stack([g, g + 1])


def kernel(edge_index, etype, r_to_e, rel_seg, dynamic_emb, emb_rel,
           W_ih_r, W_hh_r, b_ih_r, b_hh_r,
           W_ih_e, W_hh_e, b_ih_e, b_hh_e,
           W_n1, W_s1, W_n2, W_s2,
           time_gate_weight, time_gate_bias):
    srcv = edge_index[0]
    dstv = edge_index[1]
    dstp = jnp.pad(dstv, (0, EP - E), constant_values=N)
    segp = jnp.pad(rel_seg, (0, EP - E), constant_values=NR2) + RD
    s_cat = jnp.concatenate([dstp, segp])
    g_m = jnp.concatenate([_gidx1(srcv), _gidx1(r_to_e)],
                          axis=1).reshape(NC, NS, 2 * NCH, CH)
    s_m = s_cat.reshape(NS, 2 * NCH, CH)
    s_c = s_cat.reshape(NC, NS, NCH, CH)
    g_et = _gidx1(etype).reshape(NC, NS, NCH, CH)
    g_src = _gidx1(srcv).reshape(NC, NS, NCH, CH)
    dst_t = dstp.reshape(NS, NCH, CH)

    h = _l2norm(dynamic_emb)

    C = _scat_cnt(s_c)
    M = _scat_merged(h.reshape(2 * N, HW), g_m, s_m)
    h0 = _relgru(M[0, RD:RD + NR2], M[1, RD:RD + NR2],
                 C[0, RD:RD + NR2], C[1, RD:RD + NR2], emb_rel,
                 W_ih_r.T, W_hh_r.T, b_ih_r[None, :], b_hh_r[None, :])

    B = _scat_dst(h0.reshape(2 * NR2, HW), g_et, dst_t)

    cur = _layer(M[0], M[1], B[0], B[1], C[0], C[1], h, W_n1, W_s1)
    S = _scat_dst(cur.reshape(2 * N, HW), g_src, dst_t)
    cur = _layer(S[0], S[1], B[0], B[1], C[0], C[1], cur, W_n2, W_s2)

    return _final(cur, h, W_ih_e.T, W_hh_e.T,
                  b_ih_e[None, :], b_hh_e[None, :],
                  time_gate_weight, time_gate_bias[None, :])


# speedup vs baseline: 3.3113x; 1.1631x over previous
import functools

import jax
import jax.numpy as jnp
from jax import lax
from jax.experimental import pallas as pl
from jax.experimental.pallas import tpu as pltpu
from jax.experimental.pallas import tpu_sc as plsc

N = 10000
E = 160000
H = 256
NR2 = 200

HW = 128
NC = 2
NS = 16
CH = 64
EP = NC * NS * CH * 80
NCH = EP // NS // CH
RD = 10112
RDS = RD + 256

_SLOPE = (1.0 / 8.0 + 1.0 / 3.0) / 2.0
_EPS = 1e-12

_mesh = plsc.VectorSubcoreMesh(core_axis_name="c", subcore_axis_name="s")


def _fill(buf, nrows, width, val):
    v = jnp.full((16,), val, jnp.float32)

    def fr(j, _):
        for t in range(width // 16):
            buf[j, pl.ds(t * 16, 16)] = v
        return 0
    lax.fori_loop(0, nrows, fr, 0)


def _fill_zero(buf, nrows, width):
    _fill(buf, nrows, width, 0.0)


def _fill_one(buf, nrows, width):
    _fill(buf, nrows, width, 1.0)


def _zero_slab(slab, zbuf, s, rpt):
    full, rem = rpt // CH, rpt % CH

    def za(j, _):
        pltpu.sync_copy(zbuf, slab.at[pl.ds(s * rpt + j * CH, CH)])
        return 0
    if full:
        lax.fori_loop(0, full, za, 0)
    if rem:
        pltpu.sync_copy(zbuf.at[pl.ds(0, rem)],
                        slab.at[pl.ds(s * rpt + full * CH, rem)])


def _copy_out(slab, out, c, s, rpt):
    full, rem = rpt // CH, rpt % CH

    def ca(j, _):
        r = s * rpt + j * CH
        pltpu.sync_copy(slab.at[pl.ds(r, CH)], out.at[c, pl.ds(r, CH)])
        return 0
    if full:
        lax.fori_loop(0, full, ca, 0)
    if rem:
        r = s * rpt + full * CH
        pltpu.sync_copy(slab.at[pl.ds(r, rem)], out.at[c, pl.ds(r, rem)])


def _rowscat_body(nrows, nch, nb, *refs):
    (tbl, gidx, sidx, osum,
     gi2, si2, b0, b1, b2, b3, acc, m0, m1, m2, m3) = refs
    bufs = (b0, b1, b2, b3)
    sems = (m0, m1, m2, m3)

    c = lax.axis_index("c")
    s = lax.axis_index("s")
    rpt = nrows // NS

    _fill_zero(b0, CH, HW)
    _zero_slab(acc, b0, s, rpt)

    plsc.subcore_barrier()

    def block(l):
        pltpu.sync_copy(gidx.at[c, s, pl.ds(l * nb, nb)], gi2)
        pltpu.sync_copy(sidx.at[s, pl.ds(l * nb, nb)], si2)
        for k in range(3):
            pltpu.async_copy(tbl.at[gi2.at[k]], bufs[k], sems[k])

        def quad(j, _):
            for k in range(4):
                i = 4 * j + k
                pltpu.make_async_copy(tbl.at[gi2.at[i]],
                                      bufs[k], sems[k]).wait()

                @pl.when(i + 3 < nb)
                def _(i=i, k=k):
                    kn = (k + 3) % 4
                    pltpu.async_copy(tbl.at[gi2.at[i + 3]],
                                     bufs[kn], sems[kn])
                pltpu.sync_copy(bufs[k], acc.at[si2.at[i]], add=True)
            return 0
        lax.fori_loop(0, nb // 4, quad, 0)

    for l in range(nch // nb):
        block(l)

    plsc.subcore_barrier()
    _copy_out(acc, osum, c, s, rpt)


def _make_rowscat(nrows, nch, nb):
    scratch = [
        pltpu.VMEM((nb, CH), jnp.int32),
        pltpu.VMEM((nb, CH), jnp.int32),
        pltpu.VMEM((CH, HW), jnp.float32),
        pltpu.VMEM((CH, HW), jnp.float32),
        pltpu.VMEM((CH, HW), jnp.float32),
        pltpu.VMEM((CH, HW), jnp.float32),
        pltpu.VMEM_SHARED((nrows, HW), jnp.float32),
        pltpu.SemaphoreType.DMA,
        pltpu.SemaphoreType.DMA,
        pltpu.SemaphoreType.DMA,
        pltpu.SemaphoreType.DMA,
    ]

    @jax.jit
    def run(tbl, gidx, sidx):
        k = pl.kernel(
            functools.partial(_rowscat_body, nrows, nch, nb),
            out_type=jax.ShapeDtypeStruct((NC, nrows, HW), jnp.float32),
            mesh=_mesh,
            scratch_types=scratch,
        )
        return k(tbl, gidx, sidx)
    return run


_scat_merged = _make_rowscat(RDS, 2 * NCH, 40)
_scat_dst = _make_rowscat(RD, NCH, 40)


def _c2scat_body(nb, *refs):
    (tbl, gidx, sidx, ridx, osum,
     gi2, si2, ri2, b0, b1, b2, b3, obuf, acc, m0, m1, m2, m3) = refs
    bufs = (b0, b1, b2, b3)
    sems = (m0, m1, m2, m3)

    c = lax.axis_index("c")
    s = lax.axis_index("s")
    rpt = RDS // NS

    _fill_zero(b0, CH, HW)
    _zero_slab(acc, b0, s, rpt)
    _fill_one(obuf, CH, HW)

    plsc.subcore_barrier()

    def block(l):
        pltpu.sync_copy(gidx.at[c, s, pl.ds(l * nb, nb)], gi2)
        pltpu.sync_copy(sidx.at[s, pl.ds(l * nb, nb)], si2)
        pltpu.sync_copy(ridx.at[s, pl.ds(l * nb, nb)], ri2)
        for k in range(3):
            pltpu.async_copy(tbl.at[gi2.at[k]], bufs[k], sems[k])

        def quad(j, _):
            for k in range(4):
                i = 4 * j + k
                pltpu.make_async_copy(tbl.at[gi2.at[i]],
                                      bufs[k], sems[k]).wait()

                @pl.when(i + 3 < nb)
                def _(i=i, k=k):
                    kn = (k + 3) % 4
                    pltpu.async_copy(tbl.at[gi2.at[i + 3]],
                                     bufs[kn], sems[kn])
                pltpu.sync_copy(bufs[k], acc.at[si2.at[i]], add=True)

                @pl.when(c == 0)
                def _(i=i):
                    pltpu.sync_copy(obuf, acc.at[ri2.at[i]], add=True)
            return 0
        lax.fori_loop(0, nb // 4, quad, 0)

    for l in range(NCH // nb):
        block(l)

    plsc.subcore_barrier()
    _copy_out(acc, osum, c, s, rpt)


@jax.jit
def _scat_c2(tbl, gidx, sidx, ridx):
    nb = 16
    k = pl.kernel(
        functools.partial(_c2scat_body, nb),
        out_type=jax.ShapeDtypeStruct((NC, RDS, HW), jnp.float32),
        mesh=_mesh,
        scratch_types=[
            pltpu.VMEM((nb, CH), jnp.int32),
            pltpu.VMEM((nb, CH), jnp.int32),
            pltpu.VMEM((nb, CH), jnp.int32),
            pltpu.VMEM((CH, HW), jnp.float32),
            pltpu.VMEM((CH, HW), jnp.float32),
            pltpu.VMEM((CH, HW), jnp.float32),
            pltpu.VMEM((CH, HW), jnp.float32),
            pltpu.VMEM((CH, HW), jnp.float32),
            pltpu.VMEM_SHARED((RDS, HW), jnp.float32),
            pltpu.SemaphoreType.DMA,
            pltpu.SemaphoreType.DMA,
            pltpu.SemaphoreType.DMA,
            pltpu.SemaphoreType.DMA,
        ],
    )
    return k(tbl, gidx, sidx, ridx)


_BR = 1000


def _l2norm_body(x_ref, o_ref):
    x = x_ref[...]
    n = jnp.sqrt(jnp.sum(x * x, axis=1, keepdims=True))
    o_ref[...] = x / jnp.maximum(n, _EPS)


def _l2norm(x):
    return pl.pallas_call(
        _l2norm_body,
        grid=(N // _BR,),
        in_specs=[pl.BlockSpec((_BR, H), lambda i: (i, 0))],
        out_specs=pl.BlockSpec((_BR, H), lambda i: (i, 0)),
        out_shape=jax.ShapeDtypeStruct((N, H), jnp.float32),
    )(x)


def _relgru_body(s0_ref, s1_ref, c0_ref, emb_ref,
                 wih_ref, whh_ref, bih_ref, bhh_ref, o_ref):
    s0 = s0_ref[...]
    s1 = s1_ref[...]
    cnt = jnp.maximum(c0_ref[...][:, 0:1], 1.0)
    x0 = s0 / cnt
    x1 = s1 / cnt
    emb = emb_ref[...]
    wih = wih_ref[...]
    gi = (jnp.dot(emb, wih[:H], preferred_element_type=jnp.float32)
          + jnp.dot(x0, wih[H:H + HW], preferred_element_type=jnp.float32)
          + jnp.dot(x1, wih[H + HW:], preferred_element_type=jnp.float32)
          + bih_ref[...])
    gh = jnp.dot(emb, whh_ref[...],
                 preferred_element_type=jnp.float32) + bhh_ref[...]
    r = jax.nn.sigmoid(gi[:, :H] + gh[:, :H])
    z = jax.nn.sigmoid(gi[:, H:2 * H] + gh[:, H:2 * H])
    n = jnp.tanh(gi[:, 2 * H:] + r * gh[:, 2 * H:])
    out = (1.0 - z) * n + z * emb
    nrm = jnp.sqrt(jnp.sum(out * out, axis=1, keepdims=True))
    o_ref[...] = out / jnp.maximum(nrm, _EPS)


def _relgru(s0, s1, c0, emb_rel, wihT, whhT, bih, bhh):
    return pl.pallas_call(
        _relgru_body,
        grid=(1,),
        in_specs=[
            pl.BlockSpec((NR2, HW), lambda k: (0, 0)),
            pl.BlockSpec((NR2, HW), lambda k: (0, 0)),
            pl.BlockSpec((NR2, HW), lambda k: (0, 0)),
            pl.BlockSpec((NR2, H), lambda k: (0, 0)),
            pl.BlockSpec((2 * H, 3 * H), lambda k: (0, 0)),
            pl.BlockSpec((H, 3 * H), lambda k: (0, 0)),
            pl.BlockSpec((1, 3 * H), lambda k: (0, 0)),
            pl.BlockSpec((1, 3 * H), lambda k: (0, 0)),
        ],
        out_specs=pl.BlockSpec((NR2, H), lambda k: (0, 0)),
        out_shape=jax.ShapeDtypeStruct((NR2, H), jnp.float32),
    )(s0, s1, c0, emb_rel, wihT, whhT, bih, bhh)


def _layer_body(s0_ref, s1_ref, c2a_ref, c2b_ref, h0p_ref,
                cur_ref, wn_ref, ws_ref, o_ref):
    c2a = c2a_ref[...]
    c2b = c2b_ref[...]
    deg = (jnp.sum(c2a, axis=1, keepdims=True)
           + jnp.sum(c2b, axis=1, keepdims=True))
    inv = 1.0 / jnp.maximum(deg, 1.0)
    h0p = h0p_ref[...]
    b = (jnp.dot(c2a, h0p[:HW], preferred_element_type=jnp.float32)
         + jnp.dot(c2b, h0p[HW:], preferred_element_type=jnp.float32))
    e0 = (s0_ref[...] + b[:, :HW]) * inv
    e1 = (s1_ref[...] + b[:, HW:]) * inv
    wn = wn_ref[...]
    y = (jnp.dot(e0, wn[:HW], preferred_element_type=jnp.float32)
         + jnp.dot(e1, wn[HW:], preferred_element_type=jnp.float32)
         + jnp.dot(cur_ref[...], ws_ref[...],
                   preferred_element_type=jnp.float32))
    o_ref[...] = jnp.where(y >= 0, y, _SLOPE * y)


def _layer(s0, s1, c2a, c2b, h0p, cur, W_n, W_s):
    rspec = pl.BlockSpec((_BR, HW), lambda i: (i, 0))
    return pl.pallas_call(
        _layer_body,
        grid=(N // _BR,),
        in_specs=[
            rspec, rspec, rspec, rspec,
            pl.BlockSpec((H, H), lambda i: (0, 0)),
            pl.BlockSpec((_BR, H), lambda i: (i, 0)),
            pl.BlockSpec((H, H), lambda i: (0, 0)),
            pl.BlockSpec((H, H), lambda i: (0, 0)),
        ],
        out_specs=pl.BlockSpec((_BR, H), lambda i: (i, 0)),
        out_shape=jax.ShapeDtypeStruct((N, H), jnp.float32),
    )(s0, s1, c2a, c2b, h0p, cur, W_n, W_s)


def _final_body(cur_ref, h_ref, wih_ref, whh_ref, bih_ref, bhh_ref,
                wt_ref, bt_ref, o_ref):
    cur = cur_ref[...]
    nrm = jnp.sqrt(jnp.sum(cur * cur, axis=1, keepdims=True))
    x = cur / jnp.maximum(nrm, _EPS)
    hb = h_ref[...]
    gi = jnp.dot(x, wih_ref[...],
                 preferred_element_type=jnp.float32) + bih_ref[...]
    gh = jnp.dot(hb, whh_ref[...],
                 preferred_element_type=jnp.float32) + bhh_ref[...]
    r = jax.nn.sigmoid(gi[:, :H] + gh[:, :H])
    z = jax.nn.sigmoid(gi[:, H:2 * H] + gh[:, H:2 * H])
    n = jnp.tanh(gi[:, 2 * H:] + r * gh[:, 2 * H:])
    hn = (1.0 - z) * n + z * hb
    nrm2 = jnp.sqrt(jnp.sum(hn * hn, axis=1, keepdims=True))
    hn = hn / jnp.maximum(nrm2, _EPS)
    gate = jax.nn.sigmoid(
        jnp.dot(hn, wt_ref[...], preferred_element_type=jnp.float32)
        + bt_ref[...])
    o_ref[...] = gate * hn + (1.0 - gate) * hb


def _final(cur, h, wihT, whhT, bih, bhh, wt, bt):
    return pl.pallas_call(
        _final_body,
        grid=(N // _BR,),
        in_specs=[
            pl.BlockSpec((_BR, H), lambda i: (i, 0)),
            pl.BlockSpec((_BR, H), lambda i: (i, 0)),
            pl.BlockSpec((H, 3 * H), lambda i: (0, 0)),
            pl.BlockSpec((H, 3 * H), lambda i: (0, 0)),
            pl.BlockSpec((1, 3 * H), lambda i: (0, 0)),
            pl.BlockSpec((1, 3 * H), lambda i: (0, 0)),
            pl.BlockSpec((H, H), lambda i: (0, 0)),
            pl.BlockSpec((1, H), lambda i: (0, 0)),
        ],
        out_specs=pl.BlockSpec((_BR, H), lambda i: (i, 0)),
        out_shape=jax.ShapeDtypeStruct((N, H), jnp.float32),
    )(cur, h, wihT, whhT, bih, bhh, wt, bt)


def _gidx1(idx):
    g = jnp.pad(2 * idx, (0, EP - E))
    return jnp.stack([g, g + 1])


def kernel(edge_index, etype, r_to_e, rel_seg, dynamic_emb, emb_rel,
           W_ih_r, W_hh_r, b_ih_r, b_hh_r,
           W_ih_e, W_hh_e, b_ih_e, b_hh_e,
           W_n1, W_s1, W_n2, W_s2,
           time_gate_weight, time_gate_bias):
    srcv = edge_index[0]
    dstv = edge_index[1]
    dstp = jnp.pad(dstv, (0, EP - E), constant_values=N)
    segp = jnp.pad(rel_seg, (0, EP - E), constant_values=NR2) + RD
    g_m = jnp.concatenate([_gidx1(srcv), _gidx1(r_to_e)],
                          axis=1).reshape(NC, NS, 2 * NCH, CH)
    s_m = jnp.concatenate([dstp, segp]).reshape(NS, 2 * NCH, CH)
    g_et = _gidx1(etype).reshape(NC, NS, NCH, CH)
    g_src = _gidx1(srcv).reshape(NC, NS, NCH, CH)
    dst_t = dstp.reshape(NS, NCH, CH)
    seg_t = segp.reshape(NS, NCH, CH)

    eye2 = jnp.pad(jnp.eye(NR2, dtype=jnp.float32),
                   ((0, 0), (0, 2 * HW - NR2))).reshape(2 * NR2, HW)

    h = _l2norm(dynamic_emb)

    K = _scat_c2(eye2, g_et, dst_t, seg_t)
    M = _scat_merged(h.reshape(2 * N, HW), g_m, s_m)
    h0 = _relgru(M[0, RD:RD + NR2], M[1, RD:RD + NR2],
                 K[0, RD:RD + NR2], emb_rel,
                 W_ih_r.T, W_hh_r.T, b_ih_r[None, :], b_hh_r[None, :])
    h0p = jnp.pad(h0, ((0, 2 * HW - NR2), (0, 0)))

    cur = _layer(M[0], M[1], K[0], K[1], h0p, h, W_n1, W_s1)
    S = _scat_dst(cur.reshape(2 * N, HW), g_src, dst_t)
    cur = _layer(S[0], S[1], K[0], K[1], h0p, cur, W_n2, W_s2)

    return _final(cur, h, W_ih_e.T, W_hh_e.T,
                  b_ih_e[None, :], b_hh_e[None, :],
                  time_gate_weight, time_gate_bias[None, :])
